# baseline (speedup 1.0000x reference)
import functools

import jax
import jax.numpy as jnp
from jax import lax
from jax.experimental import pallas as pl
from jax.experimental.pallas import tpu as pltpu
from jax.experimental.pallas import tpu_sc as plsc

B = 16384
D = 64
NC = 2
NS = 16
NW = NC * NS
L = 16

B_TC = 4096
B_SC = B - B_TC
B_PER_W = B_SC // NW
N_GROUPS = B_PER_W // L


def _start_pair(uet_hbm, iet_hbm, ue_bufs, ie_bufs, sems, ucol, icol, p):
    cu = pltpu.make_async_copy(uet_hbm.at[:, pl.ds(ucol, 128)],
                               ue_bufs[p], sems[2 * p])
    ci = pltpu.make_async_copy(iet_hbm.at[:, pl.ds(icol, 128)],
                               ie_bufs[p], sems[2 * p + 1])
    cu.start()
    ci.start()


def _wait_pair(uet_hbm, iet_hbm, ue_bufs, ie_bufs, sems, p):
    pltpu.make_async_copy(uet_hbm.at[:, pl.ds(0, 128)],
                          ue_bufs[p], sems[2 * p]).wait()
    pltpu.make_async_copy(iet_hbm.at[:, pl.ds(0, 128)],
                          ie_bufs[p], sems[2 * p + 1]).wait()


def _body(u_hbm, i_hbm, uet_hbm, iet_hbm, out_hbm,
          u_idx, i_idx, ue_t0, ue_t1, ue_t2, ue_t3,
          ie_t0, ie_t1, ie_t2, ie_t3, out_v,
          s0, s1, s2, s3, s4, s5, s6, s7):
    wid = lax.axis_index("s") * NC + lax.axis_index("c")
    base = wid * B_PER_W

    pltpu.sync_copy(u_hbm.at[pl.ds(base, B_PER_W)],
                    u_idx.at[pl.ds(0, B_PER_W)])
    pltpu.sync_copy(i_hbm.at[pl.ds(base, B_PER_W)],
                    i_idx.at[pl.ds(0, B_PER_W)])

    ue_bufs = (ue_t0, ue_t1, ue_t2, ue_t3)
    ie_bufs = (ie_t0, ie_t1, ie_t2, ie_t3)
    sems = (s0, s1, s2, s3, s4, s5, s6, s7)
    lane = lax.iota(jnp.int32, L)

    def col_of(vec16, bb):
        return pl.multiple_of((vec16[bb] >> 7) * 128, 128)

    u16_0 = u_idx[pl.ds(0, L)]
    i16_0 = i_idx[pl.ds(0, L)]
    for p0 in range(4):
        _start_pair(uet_hbm, iet_hbm, ue_bufs, ie_bufs, sems,
                    col_of(u16_0, p0), col_of(i16_0, p0), p0)

    def group(g, carry):
        gbase = g * L
        u16 = u_idx[pl.ds(gbase, L)]
        i16 = i_idx[pl.ds(gbase, L)]
        ui16 = u16 & 127
        ii16 = i16 & 127
        acc = jnp.zeros((L,), jnp.float32)
        for bb in range(L):
            p = bb & 3
            _wait_pair(uet_hbm, iet_hbm, ue_bufs, ie_bufs, sems, p)

            ui = ui16[bb]
            ii = ii16[bb]
            uiv = jnp.full((L,), ui, jnp.int32)
            iiv = jnp.full((L,), ii, jnp.int32)
            ueb = ue_bufs[p]
            ieb = ie_bufs[p]
            prod = jnp.zeros((L,), jnp.float32)
            for c in range(D // L):
                rows = c * L + lane
                uv = plsc.load_gather(ueb, [rows, uiv])
                iv = plsc.load_gather(ieb, [rows, iiv])
                prod = prod + uv * iv
            acc = jnp.where(lane == bb, jnp.sum(prod), acc)

            nxt = gbase + bb + 4

            @pl.when(nxt < B_PER_W)
            def _():
                un = u_idx[pl.ds(nxt, L)]
                inx = i_idx[pl.ds(nxt, L)]
                _start_pair(uet_hbm, iet_hbm, ue_bufs, ie_bufs, sems,
                            col_of(un, 0), col_of(inx, 0), p)

        out_v[pl.ds(gbase, L)] = acc
        return carry

    lax.fori_loop(0, N_GROUPS, group, 0)

    pltpu.sync_copy(out_v, out_hbm.at[pl.ds(base, B_PER_W)])


def _sc_score(u, i, uet, iet):
    mesh = plsc.VectorSubcoreMesh(core_axis_name="c", subcore_axis_name="s")
    f = functools.partial(
        pl.kernel,
        out_type=jax.ShapeDtypeStruct((B_SC,), jnp.float32),
        mesh=mesh,
        compiler_params=pltpu.CompilerParams(needs_layout_passes=False),
        scratch_types=[
            pltpu.VMEM((B_PER_W + L,), jnp.int32),
            pltpu.VMEM((B_PER_W + L,), jnp.int32),
        ] + [pltpu.VMEM((D, 128), jnp.float32)] * 8
        + [pltpu.VMEM((B_PER_W,), jnp.float32)]
        + [pltpu.SemaphoreType.DMA] * 8,
    )(_body)
    return f(u, i, uet, iet)


def _tc_body(u_s, i_s, ublk, iblk, o_ref):
    b = pl.program_id(0)
    r = b % 8
    lu = u_s[b] & 127
    li = i_s[b] & 127
    iota = lax.broadcasted_iota(jnp.int32, (D, 128), 1)
    ucol = jnp.sum(jnp.where(iota == lu, ublk[...], 0.0), axis=1)
    icol = jnp.sum(jnp.where(iota == li, iblk[...], 0.0), axis=1)
    score = jnp.sum(ucol * icol)

    rows = lax.broadcasted_iota(jnp.int32, (8, 128), 0)

    @pl.when(r == 0)
    def _():
        o_ref[...] = jnp.zeros((8, 128), jnp.float32)

    o_ref[...] = jnp.where(rows == r, score, o_ref[...])


def _tc_score(u, i, uet, iet):
    grid_spec = pltpu.PrefetchScalarGridSpec(
        num_scalar_prefetch=2,
        grid=(B_TC,),
        in_specs=[
            pl.BlockSpec((D, 128), lambda b, u_s, i_s: (0, u_s[b] // 128)),
            pl.BlockSpec((D, 128), lambda b, u_s, i_s: (0, i_s[b] // 128)),
        ],
        out_specs=pl.BlockSpec((8, 128), lambda b, u_s, i_s: (b // 8, 0)),
    )
    out = pl.pallas_call(
        _tc_body,
        grid_spec=grid_spec,
        out_shape=jax.ShapeDtypeStruct((B_TC, 128), jnp.float32),
        compiler_params=pltpu.CompilerParams(
            dimension_semantics=("arbitrary",)),
    )(u, i, uet, iet)
    return out[:, 0]


@jax.jit
def _score(u, i, uet, iet):
    out_tc = _tc_score(u[:B_TC], i[:B_TC], uet, iet)
    out_sc = _sc_score(u[B_TC:], i[B_TC:], uet, iet)
    return jnp.concatenate([out_tc, out_sc])


def kernel(u, i, user_emb, item_emb):
    return _score(u, i, user_emb.T, item_emb.T)


# speedup vs baseline: 6.0821x; 6.0821x over previous
import functools

import jax
import jax.numpy as jnp
from jax import lax
from jax.experimental import pallas as pl
from jax.experimental.pallas import tpu as pltpu
from jax.experimental.pallas import tpu_sc as plsc

B = 16384
D = 64
NC = 2
NS = 16
NW = NC * NS
B_PER_W = B // NW
L = 16
N_GROUPS = B_PER_W // L


def _start_pair(uet_hbm, iet_hbm, ue_bufs, ie_bufs, sems, ucol, icol, p):
    cu = pltpu.make_async_copy(uet_hbm.at[:, pl.ds(ucol, 128)],
                               ue_bufs[p], sems[2 * p])
    ci = pltpu.make_async_copy(iet_hbm.at[:, pl.ds(icol, 128)],
                               ie_bufs[p], sems[2 * p + 1])
    cu.start()
    ci.start()


def _wait_pair(uet_hbm, iet_hbm, ue_bufs, ie_bufs, sems, p):
    pltpu.make_async_copy(uet_hbm.at[:, pl.ds(0, 128)],
                          ue_bufs[p], sems[2 * p]).wait()
    pltpu.make_async_copy(iet_hbm.at[:, pl.ds(0, 128)],
                          ie_bufs[p], sems[2 * p + 1]).wait()


def _body(u_hbm, i_hbm, uet_hbm, iet_hbm, out_hbm,
          u_idx, i_idx, ue_t0, ue_t1, ue_t2, ue_t3,
          ie_t0, ie_t1, ie_t2, ie_t3, out_v,
          s0, s1, s2, s3, s4, s5, s6, s7):
    wid = lax.axis_index("s") * NC + lax.axis_index("c")
    base = wid * B_PER_W

    pltpu.sync_copy(u_hbm.at[pl.ds(base, B_PER_W)],
                    u_idx.at[pl.ds(0, B_PER_W)])
    pltpu.sync_copy(i_hbm.at[pl.ds(base, B_PER_W)],
                    i_idx.at[pl.ds(0, B_PER_W)])

    ue_bufs = (ue_t0, ue_t1, ue_t2, ue_t3)
    ie_bufs = (ie_t0, ie_t1, ie_t2, ie_t3)
    sems = (s0, s1, s2, s3, s4, s5, s6, s7)
    lane = lax.iota(jnp.int32, L)

    def col_of(vec16, bb):
        return pl.multiple_of((vec16[bb] >> 7) * 128, 128)

    u16_0 = u_idx[pl.ds(0, L)]
    i16_0 = i_idx[pl.ds(0, L)]
    for p0 in range(4):
        _start_pair(uet_hbm, iet_hbm, ue_bufs, ie_bufs, sems,
                    col_of(u16_0, p0), col_of(i16_0, p0), p0)

    def group(g, carry):
        gbase = g * L
        u16 = u_idx[pl.ds(gbase, L)]
        i16 = i_idx[pl.ds(gbase, L)]
        u16n = u_idx[pl.ds(gbase + L, L)]
        i16n = i_idx[pl.ds(gbase + L, L)]
        ui16 = u16 & 127
        ii16 = i16 & 127
        acc = jnp.zeros((L,), jnp.float32)
        for bb in range(L):
            p = bb & 3
            _wait_pair(uet_hbm, iet_hbm, ue_bufs, ie_bufs, sems, p)

            ui = ui16[bb]
            ii = ii16[bb]
            uiv = jnp.full((L,), ui, jnp.int32)
            iiv = jnp.full((L,), ii, jnp.int32)
            ueb = ue_bufs[p]
            ieb = ie_bufs[p]
            prod = jnp.zeros((L,), jnp.float32)
            for c in range(D // L):
                rows = c * L + lane
                uv = plsc.load_gather(ueb, [rows, uiv])
                iv = plsc.load_gather(ieb, [rows, iiv])
                prod = prod + uv * iv
            acc = jnp.where(lane == bb, jnp.sum(prod), acc)

            nxt = gbase + bb + 4
            if bb < L - 4:
                ucol_n = col_of(u16, bb + 4)
                icol_n = col_of(i16, bb + 4)
            else:
                ucol_n = col_of(u16n, bb - (L - 4))
                icol_n = col_of(i16n, bb - (L - 4))

            @pl.when(nxt < B_PER_W)
            def _():
                _start_pair(uet_hbm, iet_hbm, ue_bufs, ie_bufs, sems,
                            ucol_n, icol_n, p)

        out_v[pl.ds(gbase, L)] = acc
        return carry

    lax.fori_loop(0, N_GROUPS, group, 0)

    pltpu.sync_copy(out_v, out_hbm.at[pl.ds(base, B_PER_W)])


@jax.jit
def _score(u, i, uet, iet):
    mesh = plsc.VectorSubcoreMesh(core_axis_name="c", subcore_axis_name="s")
    f = functools.partial(
        pl.kernel,
        out_type=jax.ShapeDtypeStruct((B,), jnp.float32),
        mesh=mesh,
        compiler_params=pltpu.CompilerParams(needs_layout_passes=False),
        scratch_types=[
            pltpu.VMEM((B_PER_W + L,), jnp.int32),
            pltpu.VMEM((B_PER_W + L,), jnp.int32),
        ] + [pltpu.VMEM((D, 128), jnp.float32)] * 8
        + [pltpu.VMEM((B_PER_W,), jnp.float32)]
        + [pltpu.SemaphoreType.DMA] * 8,
    )(_body)
    return f(u, i, uet, iet)


def kernel(u, i, user_emb, item_emb):
    return _score(u, i, user_emb.T, item_emb.T)
